# baseline (speedup 1.0000x reference)
import functools

import jax
import jax.numpy as jnp
from jax import lax
from jax.experimental import pallas as pl
from jax.experimental.pallas import tpu as pltpu
from jax.experimental.pallas import tpu_sc as plsc

NC = 2
NS = 16
NW = NC * NS

S = 16384
T = 50
D = 64
SEQ_PER_W = S // NW
NBUF = 8
MAIN_G = (SEQ_PER_W - NBUF) // NBUF
assert SEQ_PER_W % NBUF == 0

_mesh = plsc.VectorSubcoreMesh(core_axis_name="c", subcore_axis_name="s")


@functools.partial(
    pl.kernel,
    out_type=jax.ShapeDtypeStruct((S, T, D), jnp.float32),
    mesh=_mesh,
    scratch_types=(
        [pltpu.VMEM((T,), jnp.int32) for _ in range(NBUF)]
        + [pltpu.VMEM((T, D), jnp.float32) for _ in range(NBUF)]
        + [pltpu.SemaphoreType.DMA for _ in range(2 * NBUF)]
    ),
    compiler_params=pltpu.CompilerParams(use_tc_tiling_on_sc=False),
)
def _gather_kernel(ids_hbm, table_hbm, out_hbm, *scratch):
    idx_bufs = scratch[:NBUF]
    row_bufs = scratch[NBUF:2 * NBUF]
    gsems = scratch[2 * NBUF:3 * NBUF]
    osems = scratch[3 * NBUF:]

    wid = lax.axis_index("s") * NC + lax.axis_index("c")
    wbase = wid * SEQ_PER_W

    for b in range(NBUF):
        pltpu.sync_copy(ids_hbm.at[wbase + b], idx_bufs[b])
        pltpu.async_copy(table_hbm.at[idx_bufs[b]], row_bufs[b], gsems[b])

    @pl.loop(0, MAIN_G)
    def main(g):
        for b in range(NBUF):
            i = wbase + g * NBUF + b
            pltpu.make_async_copy(
                table_hbm.at[idx_bufs[b]], row_bufs[b], gsems[b]).wait()
            pltpu.async_copy(row_bufs[b], out_hbm.at[i], osems[b])
            pltpu.sync_copy(ids_hbm.at[i + NBUF], idx_bufs[b])
            pltpu.make_async_copy(
                row_bufs[b], out_hbm.at[i], osems[b]).wait()
            pltpu.async_copy(table_hbm.at[idx_bufs[b]], row_bufs[b], gsems[b])

    for b in range(NBUF):
        i = wbase + MAIN_G * NBUF + b
        pltpu.make_async_copy(
            table_hbm.at[idx_bufs[b]], row_bufs[b], gsems[b]).wait()
        pltpu.async_copy(row_bufs[b], out_hbm.at[i], osems[b])
    for b in range(NBUF):
        i = wbase + MAIN_G * NBUF + b
        pltpu.make_async_copy(
            row_bufs[b], out_hbm.at[i], osems[b]).wait()


def kernel(token_ids, weight):
    return _gather_kernel(token_ids.astype(jnp.int32), weight)
